# baseline (speedup 1.0000x reference)
import functools

import jax
import jax.numpy as jnp
from jax import lax
from jax.experimental import pallas as pl
from jax.experimental.pallas import tpu as pltpu
from jax.experimental.pallas import tpu_sc as plsc

_CHUNK = 128
_UCHUNKS = 2
_RING = 2
_SUPER = 4


def _sc_gather(table3d, idx2d, nb_sc):
    nb, ntab, d = table3d.shape
    n_chunks_all, chunk = idx2d.shape
    rows_all = n_chunks_all * chunk
    rpb = rows_all // nb
    urows = _UCHUNKS * chunk
    chunks_per_b = rpb // chunk
    units_per_b = chunks_per_b // _UCHUNKS
    info = plsc.get_sparse_core_info()
    nc, ns = info.num_cores, info.num_subcores
    nw = nc * ns
    b_per_w = nb_sc // nw
    chunks_per_w = b_per_w * chunks_per_b
    rows_per_w = chunks_per_w * chunk
    n_units = chunks_per_w // _UCHUNKS
    mesh = plsc.VectorSubcoreMesh(core_axis_name="c", subcore_axis_name="s")

    @functools.partial(
        pl.kernel,
        mesh=mesh,
        out_type=jax.ShapeDtypeStruct((rows_all, d), jnp.float32),
        scratch_types=[
            pltpu.VMEM((chunks_per_w, chunk), jnp.int32),
        ]
        + [pltpu.VMEM((urows, d), jnp.float32) for _ in range(_RING)]
        + [pltpu.VMEM_SHARED((ns, 2, ntab, d), jnp.float32)]
        + [pltpu.SemaphoreType.DMA for _ in range(8)],
    )
    def k(table_hbm, idx_hbm, out_hbm, idx_v, *rest):
        ubufs = rest[:_RING]
        spm = rest[_RING]
        gsems = [rest[3 + 2 * r: 5 + 2 * r] for r in range(_RING)]
        wsems = rest[7:9]
        tsems = rest[9:11]
        cid = lax.axis_index("c")
        sid = lax.axis_index("s")
        wid = sid * nc + cid
        cbase = wid * chunks_per_w
        base = wid * rows_per_w
        bbase = wid * b_per_w

        def stage(bl, slot):
            return pltpu.make_async_copy(
                table_hbm.at[bbase + bl], spm.at[sid, slot], tsems[slot])

        def gather(c, r, h, slot):
            return pltpu.make_async_copy(
                spm.at[sid, slot].at[idx_v.at[c]],
                ubufs[r].at[pl.ds(h * chunk, chunk)], gsems[r][h])

        def write(u, r):
            return pltpu.make_async_copy(
                ubufs[r], out_hbm.at[pl.ds(base + u * urows, urows)],
                wsems[r])

        def unit(u, t):
            r = t % _RING
            sslot = (t // units_per_b) % 2
            bl = u // units_per_b
            if t % units_per_b == 0:
                stage(bl, sslot).wait()

            @pl.when(u >= _RING)
            def _():
                write(u - _RING, r).wait()

            for h in range(_UCHUNKS):
                gather(u * _UCHUNKS + h, r, h, sslot).start()
            for h in range(_UCHUNKS):
                gather(u * _UCHUNKS + h, r, h, sslot).wait()
            write(u, r).start()
            if t % units_per_b == units_per_b - 1:
                @pl.when(bl + 2 < b_per_w)
                def _():
                    stage(bl + 2, sslot).start()

        stage(0, 0).start()
        stage(1, 1).start()
        pltpu.sync_copy(idx_hbm.at[pl.ds(cbase, chunks_per_w)], idx_v)

        def body(i, carry):
            for t in range(_SUPER):
                unit(i * _SUPER + t, t)
            return carry

        lax.fori_loop(0, n_units // _SUPER, body, 0)
        for t in range(_RING):
            u = n_units - _RING + t
            write(u, (u % _SUPER) % _RING).wait()

    return k(table3d, idx2d)


def _tc_fill(full_out, table3d, pairs3, nb_sc):
    nb, ntab, d = table3d.shape
    nb_tc = nb - nb_sc
    rpb = pairs3.shape[2]
    out3 = full_out.reshape(nb, rpb, d)

    def body(prev_ref, er_ref, pr_ref, out_ref):
        pj = pr_ref[0, 0]
        onehot = (
            pj[:, None]
            == lax.broadcasted_iota(jnp.int32, (rpb, ntab), 1)
        ).astype(jnp.float32)
        out_ref[0] = lax.dot_general(
            onehot, er_ref[0], (((1,), (0,)), ((), ())),
            precision=lax.Precision.HIGHEST)

    return pl.pallas_call(
        body,
        grid=(nb_tc,),
        in_specs=[
            pl.BlockSpec(memory_space=pl.MemorySpace.ANY),
            pl.BlockSpec((1, ntab, d), lambda b: (nb_sc + b, 0, 0)),
            pl.BlockSpec((1, 1, rpb), lambda b: (nb_sc + b, 0, 0)),
        ],
        out_specs=pl.BlockSpec((1, rpb, d), lambda b: (nb_sc + b, 0, 0)),
        out_shape=jax.ShapeDtypeStruct((nb, rpb, d), jnp.float32),
        input_output_aliases={0: 0},
    )(out3, table3d, pairs3)


def kernel(entity_reprs, pairs):
    b, n, d = entity_reprs.shape
    p = pairs.shape[1]
    rpb = p * 2
    nb_sc = b // 2
    pairs_i = pairs.astype(jnp.int32)
    idx = pairs_i.reshape(b * rpb // _CHUNK, _CHUNK)
    out = _sc_gather(entity_reprs, idx, nb_sc)
    out = _tc_fill(out, entity_reprs, pairs_i.reshape(b, 1, rpb), nb_sc)
    return out.reshape(b, p, 2 * d)


# speedup vs baseline: 1.5158x; 1.5158x over previous
import functools

import jax
import jax.numpy as jnp
from jax import lax
from jax.experimental import pallas as pl
from jax.experimental.pallas import tpu as pltpu
from jax.experimental.pallas import tpu_sc as plsc

_CHUNK = 128
_UCHUNKS = 2
_RING = 2
_SUPER = 4


def _sc_gather(table3d, idx2d):
    nb, ntab, d = table3d.shape
    n_chunks, chunk = idx2d.shape
    rows = n_chunks * chunk
    urows = _UCHUNKS * chunk
    chunks_per_b = rows // nb // chunk
    units_per_b = chunks_per_b // _UCHUNKS
    info = plsc.get_sparse_core_info()
    nc, ns = info.num_cores, info.num_subcores
    nw = nc * ns
    chunks_per_w = n_chunks // nw
    rows_per_w = chunks_per_w * chunk
    b_per_w = nb // nw
    n_units = chunks_per_w // _UCHUNKS
    mesh = plsc.VectorSubcoreMesh(core_axis_name="c", subcore_axis_name="s")

    @functools.partial(
        pl.kernel,
        mesh=mesh,
        out_type=jax.ShapeDtypeStruct((rows, d), jnp.float32),
        scratch_types=[
            pltpu.VMEM((chunks_per_w, chunk), jnp.int32),
        ]
        + [pltpu.VMEM((urows, d), jnp.float32) for _ in range(_RING)]
        + [pltpu.VMEM_SHARED((ns, 2, ntab, d), jnp.float32)]
        + [pltpu.SemaphoreType.DMA for _ in range(8)],
    )
    def k(table_hbm, idx_hbm, out_hbm, idx_v, *rest):
        ubufs = rest[:_RING]
        spm = rest[_RING]
        gsems = [rest[3 + 2 * r: 5 + 2 * r] for r in range(_RING)]
        wsems = rest[7:9]
        tsems = rest[9:11]
        cid = lax.axis_index("c")
        sid = lax.axis_index("s")
        wid = sid * nc + cid
        cbase = wid * chunks_per_w
        base = wid * rows_per_w
        bbase = wid * b_per_w

        def stage(bl, slot):
            return pltpu.make_async_copy(
                table_hbm.at[bbase + bl], spm.at[sid, slot], tsems[slot])

        def gather(c, r, h, slot):
            return pltpu.make_async_copy(
                spm.at[sid, slot].at[idx_v.at[c]],
                ubufs[r].at[pl.ds(h * chunk, chunk)], gsems[r][h])

        def write(u, r):
            return pltpu.make_async_copy(
                ubufs[r], out_hbm.at[pl.ds(base + u * urows, urows)],
                wsems[r])

        def unit(u, t):
            r = t % _RING
            sslot = (t // units_per_b) % 2
            bl = u // units_per_b
            if t % units_per_b == 0:
                stage(bl, sslot).wait()

            @pl.when(u >= _RING)
            def _():
                write(u - _RING, r).wait()

            for h in range(_UCHUNKS):
                gather(u * _UCHUNKS + h, r, h, sslot).start()
            for h in range(_UCHUNKS):
                gather(u * _UCHUNKS + h, r, h, sslot).wait()
            write(u, r).start()
            if t % units_per_b == units_per_b - 1:
                @pl.when(bl + 2 < b_per_w)
                def _():
                    stage(bl + 2, sslot).start()

        stage(0, 0).start()
        stage(1, 1).start()
        pltpu.sync_copy(idx_hbm.at[pl.ds(cbase, chunks_per_w)], idx_v)

        def body(i, carry):
            for t in range(_SUPER):
                unit(i * _SUPER + t, t)
            return carry

        lax.fori_loop(0, n_units // _SUPER, body, 0)
        for t in range(_RING):
            u = n_units - _RING + t
            write(u, (u % _SUPER) % _RING).wait()

    return k(table3d, idx2d)


def kernel(entity_reprs, pairs):
    b, n, d = entity_reprs.shape
    p = pairs.shape[1]
    idx = pairs.astype(jnp.int32).reshape(b * p * 2 // _CHUNK, _CHUNK)
    out = _sc_gather(entity_reprs, idx)
    return out.reshape(b, p, 2 * d)
